# Initial kernel scaffold; baseline (speedup 1.0000x reference)
import jax
import jax.numpy as jnp
from jax.experimental import pallas as pl


def kernel(X, W):
    raise NotImplementedError("write your pallas kernel here")


# speedup vs baseline: 1.1067x; 1.1067x over previous
import numpy as np
import jax
import jax.numpy as jnp
from jax.experimental import pallas as pl
from jax.experimental.pallas import tpu as pltpu

_B, _IN_CH, _T = 32, 4, 16384
_OUT_CH, _K = 32, 1023
_NE, _ED = 1024, 1023
_COMMIT = 0.25
_NROWS = _B * _OUT_CH * _IN_CH
_BM = 256


def _extract(X):
    t = X.shape[-1]
    padded = int(np.ceil(t / _K) * _K)
    end = padded - _K - 1 - _K
    positions = jnp.linspace(0.0, float(end), _OUT_CH).astype(jnp.int32)
    idx = positions[:, None] + jnp.arange(_K, dtype=jnp.int32)[None, :]
    filt = X[:, :, idx]
    return jnp.transpose(filt, (0, 2, 1, 3))


def _acov(f):
    eps = jnp.finfo(f.dtype).eps
    n = f.shape[-1]
    fmax = jnp.max(jnp.abs(f), axis=-1, keepdims=True)
    fmax = jnp.where(fmax == 0, eps, fmax)
    win = 0.5 * (1.0 - jnp.cos(2.0 * jnp.pi * jnp.arange(n, dtype=f.dtype) / n))
    wd = f * win / fmax
    spec = jnp.fft.rfft(wd, n=n) ** 2
    acov = jnp.fft.ifftshift(jnp.fft.irfft(jnp.abs(spec), n=n))
    return acov.astype(f.dtype)


def _vq_kernel(f_ref, w_ref, loss_ref, q_ref, perp_ref, cnt_ref, dsum_ref):
    i = pl.program_id(0)
    f = f_ref[...]
    w = w_ref[...]
    a = jnp.sum(f * f, axis=1, keepdims=True)
    b = jnp.sum(w * w, axis=1)
    mm = jax.lax.dot_general(f, w, (((1,), (1,)), ((), ())),
                             preferred_element_type=jnp.float32)
    d = (a + b[None, :]) - 2.0 * mm
    m = jnp.min(d, axis=1, keepdims=True)
    jcol = jax.lax.broadcasted_iota(jnp.int32, d.shape, 1)
    idx = jnp.min(jnp.where(d == m, jcol, jnp.int32(2 ** 30)), axis=1)
    oh = jnp.where(jcol == idx[:, None], 1.0, 0.0).astype(jnp.float32)
    q = jax.lax.dot_general(oh, w, (((1,), (0,)), ((), ())),
                            preferred_element_type=jnp.float32)
    q_ref[...] = f + (q - f)

    blk_cnt = jnp.sum(oh, axis=0, keepdims=True)
    blk_dsum = jnp.reshape(jnp.sum(m), (1, 1))

    @pl.when(i == 0)
    def _():
        cnt_ref[...] = blk_cnt
        dsum_ref[...] = blk_dsum

    @pl.when(i > 0)
    def _():
        cnt_ref[...] = cnt_ref[...] + blk_cnt
        dsum_ref[...] = dsum_ref[...] + blk_dsum

    @pl.when(i == (_NROWS // _BM) - 1)
    def _():
        mean_d = dsum_ref[...] / jnp.float32(_NROWS * _ED)
        loss_ref[...] = mean_d + _COMMIT * mean_d
        p = cnt_ref[...] / jnp.float32(_NROWS)
        feps = jnp.finfo(jnp.float32).eps
        ent = -jnp.sum(p * jnp.log(p + feps))
        perp_ref[...] = jnp.reshape(jnp.exp(ent), (1, 1))


def kernel(X, W):
    filters = _acov(_extract(X))
    flat = filters.reshape(-1, _ED)
    nblk = _NROWS // _BM
    loss, q, perp = pl.pallas_call(
        _vq_kernel,
        grid=(nblk,),
        in_specs=[pl.BlockSpec((_BM, _ED), lambda i: (i, 0)),
                  pl.BlockSpec((_NE, _ED), lambda i: (0, 0))],
        out_specs=[pl.BlockSpec((1, 1), lambda i: (0, 0)),
                   pl.BlockSpec((_BM, _ED), lambda i: (i, 0)),
                   pl.BlockSpec((1, 1), lambda i: (0, 0))],
        out_shape=[jax.ShapeDtypeStruct((1, 1), jnp.float32),
                   jax.ShapeDtypeStruct((_NROWS, _ED), jnp.float32),
                   jax.ShapeDtypeStruct((1, 1), jnp.float32)],
        scratch_shapes=[pltpu.VMEM((1, _NE), jnp.float32),
                        pltpu.VMEM((1, 1), jnp.float32)],
        compiler_params=pltpu.CompilerParams(
            dimension_semantics=("arbitrary",)),
    )(flat, W)
    return loss[0, 0], q.reshape(filters.shape), perp[0, 0]
